# Initial kernel scaffold; baseline (speedup 1.0000x reference)
import jax
import jax.numpy as jnp
from jax.experimental import pallas as pl


def kernel(features, coors, batch_size):
    raise NotImplementedError("write your pallas kernel here")


# speedup vs baseline: 1.3767x; 1.3767x over previous
import functools

import jax
import jax.numpy as jnp
from jax import lax
from jax.experimental import pallas as pl
from jax.experimental.pallas import tpu as pltpu
from jax.experimental.pallas import tpu_sc as plsc

N = 100000
C = 128
BATCH = 64
OY, OX = 32, 32
NOUT = BATCH * OY * OX
NC, NS = 2, 16
NW = NC * NS

PTS_W = N // NW
KPAD = 3136
CO_W = 3 * PTS_W
CBUF = 9392
SENT = 1 << 20

ROWS = 512
NPASS = NOUT // (ROWS * NW)
NKEYS = NW * KPAD
KCH = 6272
NCHUNK = NKEYS // KCH
G = 128

_mesh = plsc.VectorSubcoreMesh(
    core_axis_name="c", subcore_axis_name="s", num_cores=NC, num_subcores=NS)
_sc_params = pltpu.CompilerParams(needs_layout_passes=False)


def _wid():
  return lax.axis_index("s") * NC + lax.axis_index("c")


@functools.partial(
    pl.kernel,
    out_type=jax.ShapeDtypeStruct((NKEYS,), jnp.int32),
    mesh=_mesh,
    scratch_types=[
        pltpu.VMEM((CBUF,), jnp.int32),
        pltpu.VMEM((KPAD,), jnp.int32),
    ],
    compiler_params=_sc_params,
)
def _keys_kernel(coors_hbm, keys_hbm, cbuf, kbuf):
  w = _wid()
  start = w * CO_W
  astart = (start // 8) * 8
  r = start - astart
  pltpu.sync_copy(coors_hbm.at[pl.ds(astart, CBUF)], cbuf)
  lanes = jnp.arange(16, dtype=jnp.int32)

  def body(i, _):
    p = i * 16
    valid = (p + lanes) < PTS_W
    base = r + 3 * p + lanes * 3
    idx = jnp.where(valid, base, 0)
    bcol = plsc.load_gather(cbuf, [idx])
    ycol = plsc.load_gather(cbuf, [idx + 1])
    xcol = plsc.load_gather(cbuf, [idx + 2])
    key = bcol * (OY * OX) + (ycol >> 1) * OX + (xcol >> 1)
    kbuf[pl.ds(p, 16)] = jnp.where(valid, key, SENT)
    return 0

  lax.fori_loop(0, KPAD // 16, body, 0)
  pltpu.sync_copy(kbuf, keys_hbm.at[pl.ds(w * KPAD, KPAD)])


@functools.partial(
    pl.kernel,
    out_type=jax.ShapeDtypeStruct((NOUT * C,), jnp.float32),
    mesh=_mesh,
    scratch_types=[
        pltpu.VMEM((ROWS * C,), jnp.float32),
        pltpu.VMEM((KCH,), jnp.int32),
        pltpu.VMEM((KCH + G + 16,), jnp.int32),
        pltpu.VMEM((KCH + G + 16,), jnp.int32),
        pltpu.VMEM((G,), jnp.int32),
        pltpu.VMEM((G, C), jnp.float32),
        pltpu.SemaphoreType.DMA,
    ],
    compiler_params=_sc_params,
)
def _scatter_kernel(feat_hbm, keys_hbm, out_hbm, acc, kch, mid, mkey, idxg,
                    rows, sem):
  w = _wid()
  lanes = jnp.arange(16, dtype=jnp.int32)
  neg = jnp.full((16,), -jnp.inf, dtype=jnp.float32)

  def do_pass(p, _):
    rid = p * NW + w
    lo = rid * ROWS
    hi = lo + ROWS

    def initb(i, _):
      acc[pl.ds(i * 16, 16)] = neg
      return 0

    lax.fori_loop(0, ROWS * C // 16, initb, 0)

    def do_chunk(t, _):
      pltpu.sync_copy(keys_hbm.at[pl.ds(t * KCH, KCH)], kch)

      def scan(i, cnt):
        k = kch[pl.ds(i * 16, 16)]
        m = (k >= lo) & (k < hi)
        mi = m.astype(jnp.int32)
        pos = plsc.cumsum(mi) - 1 + cnt
        q = t * KCH + i * 16 + lanes
        v = q // KPAD
        pid = q - v * (KPAD - PTS_W)
        plsc.store_scatter(mid, [pos], pid, mask=m)
        plsc.store_scatter(mkey, [pos], k, mask=m)
        return cnt + jnp.sum(mi, axis=0)

      cnt = lax.fori_loop(0, KCH // 16, scan, jnp.int32(0))

      @pl.when(cnt > 0)
      def _():
        i0 = jnp.full((16,), mid[pl.ds(0, 16)][0], dtype=jnp.int32)
        k0 = jnp.full((16,), mkey[pl.ds(0, 16)][0], dtype=jnp.int32)
        for u in range(G // 16):
          mid[pl.ds(cnt + u * 16, 16)] = i0
          mkey[pl.ds(cnt + u * 16, 16)] = k0
        nb = (cnt + G - 1) // G

        def gbatch(g, _):
          gb = g * G
          for u in range(G // 16):
            idxg[pl.ds(u * 16, 16)] = mid[pl.ds(gb + u * 16, 16)]
          pltpu.async_copy(feat_hbm.at[idxg], rows, sem).wait()
          nj = jnp.minimum(jnp.int32(G), cnt - gb)

          def rmw(j, _):
            kj = mkey[pl.ds(gb + j, 16)][0]
            off = (kj - lo) * C
            for u in range(C // 16):
              a = acc[pl.ds(off + u * 16, 16)]
              f = rows[j, pl.ds(u * 16, 16)]
              acc[pl.ds(off + u * 16, 16)] = jnp.maximum(a, f)
            return 0

          lax.fori_loop(0, nj, rmw, 0)
          return 0

        lax.fori_loop(0, nb, gbatch, 0)

      return 0

    lax.fori_loop(0, NCHUNK, do_chunk, 0)
    pltpu.sync_copy(acc, out_hbm.at[pl.ds(rid * (ROWS * C), ROWS * C)])
    return 0

  lax.fori_loop(0, NPASS, do_pass, 0)


def _fixup_body(x_ref, o_ref):
  x = x_ref[0]
  xt = x.T
  o_ref[0] = jnp.where(xt == -jnp.inf, jnp.float32(0.0), xt)


_fixup = pl.pallas_call(
    _fixup_body,
    grid=(BATCH,),
    in_specs=[pl.BlockSpec((1, OY * OX, C), lambda b: (b, 0, 0))],
    out_specs=pl.BlockSpec((1, C, OY * OX), lambda b: (b, 0, 0)),
    out_shape=jax.ShapeDtypeStruct((BATCH, C, OY * OX), jnp.float32),
)


def kernel(features, coors, batch_size):
  del batch_size
  coflat = jnp.reshape(coors.astype(jnp.int32), (-1,))
  coflat = jnp.concatenate([coflat, jnp.zeros((32,), jnp.int32)])
  keys = _keys_kernel(coflat)
  accflat = _scatter_kernel(features, keys)
  dense = _fixup(accflat.reshape(BATCH, OY * OX, C))
  return dense.reshape(BATCH, C, OY, OX)


# speedup vs baseline: 1.5929x; 1.1570x over previous
import functools

import jax
import jax.numpy as jnp
from jax import lax
from jax.experimental import pallas as pl
from jax.experimental.pallas import tpu as pltpu
from jax.experimental.pallas import tpu_sc as plsc

N = 100000
C = 128
BATCH = 64
OY, OX = 32, 32
NOUT = BATCH * OY * OX
NC, NS = 2, 16
NW = NC * NS

PTS_W = N // NW
KPAD = 3136
CO_W = 3 * PTS_W
CBUF = 9392
SENT = 1 << 20

ROWS = 512
NPASS = NOUT // (ROWS * NW)
NKEYS = NW * KPAD
KCH = 6272
NCHUNK = NKEYS // KCH
G = 128

_mesh = plsc.VectorSubcoreMesh(
    core_axis_name="c", subcore_axis_name="s", num_cores=NC, num_subcores=NS)
_sc_params = pltpu.CompilerParams(needs_layout_passes=False)


def _wid():
  return lax.axis_index("s") * NC + lax.axis_index("c")


@functools.partial(
    pl.kernel,
    out_type=jax.ShapeDtypeStruct((NKEYS,), jnp.int32),
    mesh=_mesh,
    scratch_types=[
        pltpu.VMEM((CBUF,), jnp.int32),
        pltpu.VMEM((KPAD,), jnp.int32),
    ],
    compiler_params=_sc_params,
)
def _keys_kernel(coors_hbm, keys_hbm, cbuf, kbuf):
  w = _wid()
  start = w * CO_W
  astart = (start // 8) * 8
  r = start - astart
  pltpu.sync_copy(coors_hbm.at[pl.ds(astart, CBUF)], cbuf)
  lanes = jnp.arange(16, dtype=jnp.int32)

  def body(i, _):
    p = i * 16
    valid = (p + lanes) < PTS_W
    base = r + 3 * p + lanes * 3
    idx = jnp.where(valid, base, 0)
    bcol = plsc.load_gather(cbuf, [idx])
    ycol = plsc.load_gather(cbuf, [idx + 1])
    xcol = plsc.load_gather(cbuf, [idx + 2])
    key = bcol * (OY * OX) + (ycol >> 1) * OX + (xcol >> 1)
    kbuf[pl.ds(p, 16)] = jnp.where(valid, key, SENT)
    return 0

  lax.fori_loop(0, KPAD // 16, body, 0)
  pltpu.sync_copy(kbuf, keys_hbm.at[pl.ds(w * KPAD, KPAD)])


@functools.partial(
    pl.kernel,
    out_type=jax.ShapeDtypeStruct((NOUT * C,), jnp.float32),
    mesh=_mesh,
    scratch_types=[
        pltpu.VMEM((ROWS * C,), jnp.float32),
        pltpu.VMEM((KCH,), jnp.int32),
        pltpu.VMEM((KCH,), jnp.int32),
        pltpu.VMEM((KCH + G + 16,), jnp.int32),
        pltpu.VMEM((KCH + G + 16,), jnp.int32),
        pltpu.VMEM((G,), jnp.int32),
        pltpu.VMEM((G,), jnp.int32),
        pltpu.VMEM((G, C), jnp.float32),
        pltpu.VMEM((G, C), jnp.float32),
        pltpu.VMEM_SHARED((NKEYS,), jnp.int32),
        pltpu.SemaphoreType.DMA,
        pltpu.SemaphoreType.DMA,
        pltpu.SemaphoreType.DMA,
        pltpu.SemaphoreType.DMA,
    ],
    compiler_params=_sc_params,
)
def _scatter_kernel(feat_hbm, keys_hbm, out_hbm, acc, kch0, kch1, mid, mkey,
                    idxg0, idxg1, rows0, rows1, kshared, ksem0, ksem1, gsem0,
                    gsem1):
  w = _wid()
  lanes = jnp.arange(16, dtype=jnp.int32)
  neg = jnp.full((16,), -jnp.inf, dtype=jnp.float32)
  kchs = (kch0, kch1)
  ksems = (ksem0, ksem1)
  idxgs = (idxg0, idxg1)
  rowss = (rows0, rows1)
  gsems = (gsem0, gsem1)

  @pl.when(lax.axis_index("s") == 0)
  def _():
    pltpu.sync_copy(keys_hbm, kshared)

  plsc.subcore_barrier()

  def kfetch(t, b):
    pltpu.async_copy(kshared.at[pl.ds(t * KCH, KCH)], kchs[b], ksems[b])

  def do_chunk(t, b, lo, hi):
    pltpu.make_async_copy(kshared.at[pl.ds(t * KCH, KCH)], kchs[b],
                          ksems[b]).wait()

    @pl.when(t + 1 < NCHUNK)
    def _():
      kfetch(t + 1, 1 - b)

    kch = kchs[b]

    def scan(i, cnt):
      k = kch[pl.ds(i * 16, 16)]
      m = (k >= lo) & (k < hi)
      mi = m.astype(jnp.int32)
      pos = plsc.cumsum(mi) - 1 + cnt
      q = t * KCH + i * 16 + lanes
      v = q // KPAD
      pid = q - v * (KPAD - PTS_W)
      plsc.store_scatter(mid, [pos], pid, mask=m)
      plsc.store_scatter(mkey, [pos], k, mask=m)
      return cnt + jnp.sum(mi, axis=0)

    cnt = lax.fori_loop(0, KCH // 16, scan, jnp.int32(0))

    @pl.when(cnt > 0)
    def _():
      i0 = jnp.full((16,), mid[pl.ds(0, 16)][0], dtype=jnp.int32)
      k0 = jnp.full((16,), mkey[pl.ds(0, 16)][0], dtype=jnp.int32)
      for u in range(G // 16):
        mid[pl.ds(cnt + u * 16, 16)] = i0
        mkey[pl.ds(cnt + u * 16, 16)] = k0
      nb = (cnt + G - 1) // G

      def gissue(g, gb_buf):
        for u in range(G // 16):
          idxgs[gb_buf][pl.ds(u * 16, 16)] = mid[pl.ds(g * G + u * 16, 16)]
        pltpu.async_copy(feat_hbm.at[idxgs[gb_buf]], rowss[gb_buf],
                         gsems[gb_buf])

      def rmw_batch(g, gb_buf):
        pltpu.make_async_copy(feat_hbm.at[idxgs[gb_buf]], rowss[gb_buf],
                              gsems[gb_buf]).wait()

        @pl.when(g + 1 < nb)
        def _():
          gissue(g + 1, 1 - gb_buf)

        rows = rowss[gb_buf]
        gb = g * G
        nj = jnp.minimum(jnp.int32(G), cnt - gb)

        def rmw(j, _):
          kj = mkey[pl.ds(gb + j, 16)][0]
          off = (kj - lo) * C
          for u in range(C // 16):
            a = acc[pl.ds(off + u * 16, 16)]
            f = rows[j, pl.ds(u * 16, 16)]
            acc[pl.ds(off + u * 16, 16)] = jnp.maximum(a, f)
          return 0

        lax.fori_loop(0, nj, rmw, 0)

      gissue(0, 0)

      def gpair(gg, _):
        g0 = gg * 2
        rmw_batch(g0, 0)

        @pl.when(g0 + 1 < nb)
        def _():
          rmw_batch(g0 + 1, 1)

        return 0

      lax.fori_loop(0, (nb + 1) // 2, gpair, 0)

  def do_pass(p, _):
    rid = p * NW + w
    lo = rid * ROWS
    hi = lo + ROWS

    def initb(i, _):
      for u in range(8):
        acc[pl.ds(i * 128 + u * 16, 16)] = neg
      return 0

    lax.fori_loop(0, ROWS * C // 128, initb, 0)

    kfetch(0, 0)

    def cpair(tt, _):
      t0 = tt * 2
      do_chunk(t0, 0, lo, hi)

      @pl.when(t0 + 1 < NCHUNK)
      def _():
        do_chunk(t0 + 1, 1, lo, hi)

      return 0

    lax.fori_loop(0, (NCHUNK + 1) // 2, cpair, 0)
    pltpu.sync_copy(acc, out_hbm.at[pl.ds(rid * (ROWS * C), ROWS * C)])
    return 0

  lax.fori_loop(0, NPASS, do_pass, 0)


def _fixup_body(x_ref, o_ref):
  x = x_ref[0]
  xt = x.T
  o_ref[0] = jnp.where(xt == -jnp.inf, jnp.float32(0.0), xt)


_fixup = pl.pallas_call(
    _fixup_body,
    grid=(BATCH,),
    in_specs=[pl.BlockSpec((1, OY * OX, C), lambda b: (b, 0, 0))],
    out_specs=pl.BlockSpec((1, C, OY * OX), lambda b: (b, 0, 0)),
    out_shape=jax.ShapeDtypeStruct((BATCH, C, OY * OX), jnp.float32),
)


def kernel(features, coors, batch_size):
  del batch_size
  coflat = jnp.reshape(coors.astype(jnp.int32), (-1,))
  coflat = jnp.concatenate([coflat, jnp.zeros((32,), jnp.int32)])
  keys = _keys_kernel(coflat)
  accflat = _scatter_kernel(features, keys)
  dense = _fixup(accflat.reshape(BATCH, OY * OX, C))
  return dense.reshape(BATCH, C, OY, OX)


# speedup vs baseline: 2.9299x; 1.8394x over previous
import functools

import jax
import jax.numpy as jnp
from jax import lax
from jax.experimental import pallas as pl
from jax.experimental.pallas import tpu as pltpu
from jax.experimental.pallas import tpu_sc as plsc

N = 100000
C = 128
BATCH = 64
OY, OX = 32, 32
NOUT = BATCH * OY * OX
NC, NS = 2, 16
NW = NC * NS

PTS_W = N // NW
KPAD = 3136
CO_W = 3 * PTS_W
CBUF = 9392
SENTP = 0x7FFFFFFF

ROWS = 512
NPASS = NOUT // (ROWS * NW)
NKEYS = NW * KPAD
BLK = 448
G = 128
CAP = 3456

_mesh = plsc.VectorSubcoreMesh(
    core_axis_name="c", subcore_axis_name="s", num_cores=NC, num_subcores=NS)
_sc_params = pltpu.CompilerParams(needs_layout_passes=False)


def _wid():
  return lax.axis_index("s") * NC + lax.axis_index("c")


@functools.partial(
    pl.kernel,
    out_type=[jax.ShapeDtypeStruct((NKEYS,), jnp.int32),
              jax.ShapeDtypeStruct((NW * 16,), jnp.int32)],
    mesh=_mesh,
    scratch_types=[
        pltpu.VMEM((CBUF,), jnp.int32),
        pltpu.VMEM((KPAD,), jnp.int32),
        pltpu.VMEM((KPAD,), jnp.int32),
        pltpu.VMEM((16,), jnp.int32),
    ],
    compiler_params=_sc_params,
)
def _keys_kernel(coors_hbm, pk_hbm, bounds_hbm, cbuf, tmp, kbuf, bvec):
  w = _wid()
  start = w * CO_W
  astart = (start // 8) * 8
  r = start - astart
  pltpu.sync_copy(coors_hbm.at[pl.ds(astart, CBUF)], cbuf)
  lanes = jnp.arange(16, dtype=jnp.int32)
  sent = jnp.full((16,), SENTP, jnp.int32)

  def phase_a(i, carry):
    c0, c1, c2 = carry
    p = i * 16
    valid = (p + lanes) < PTS_W
    base = r + 3 * p + lanes * 3
    idx = jnp.where(valid, base, 0)
    bcol = plsc.load_gather(cbuf, [idx])
    ycol = plsc.load_gather(cbuf, [idx + 1])
    xcol = plsc.load_gather(cbuf, [idx + 2])
    key = bcol * (OY * OX) + (ycol >> 1) * OX + (xcol >> 1)
    tmp[pl.ds(p, 16)] = jnp.where(valid, (key << 12) | (p + lanes), sent)
    bb = key >> 14
    c0 = c0 + plsc.all_reduce_population_count(valid & (bb == 0))
    c1 = c1 + plsc.all_reduce_population_count(valid & (bb == 1))
    c2 = c2 + plsc.all_reduce_population_count(valid & (bb == 2))
    return (c0, c1, c2)

  z = jnp.zeros((16,), jnp.int32)
  c0, c1, c2 = lax.fori_loop(0, KPAD // 16, phase_a, (z, z, z))
  o1 = c0[0]
  o2 = o1 + c1[0]
  o3 = o2 + c2[0]
  kbuf[pl.ds(KPAD - 16, 16)] = sent

  def phase_b(i, carry):
    pk = tmp[pl.ds(i * 16, 16)]
    bb = pk >> 26
    outs = []
    for p4 in range(4):
      m = bb == p4
      mi = m.astype(jnp.int32)
      pos = plsc.cumsum(mi) - 1 + carry[p4]
      plsc.store_scatter(kbuf, [pos], pk, mask=m)
      outs.append(carry[p4] + jnp.sum(mi, axis=0))
    return tuple(outs)

  o0f, o1f, o2f, o3f = lax.fori_loop(
      0, KPAD // 16, phase_b, (jnp.int32(0), o1, o2, o3))
  del o0f, o1f, o2f, o3f
  bv = jnp.where(lanes == 0, 0,
       jnp.where(lanes == 1, o1,
       jnp.where(lanes == 2, o2,
       jnp.where(lanes == 3, o3, PTS_W))))
  bvec[pl.ds(0, 16)] = bv.astype(jnp.int32)
  pltpu.sync_copy(kbuf, pk_hbm.at[pl.ds(w * KPAD, KPAD)])
  pltpu.sync_copy(bvec, bounds_hbm.at[pl.ds(w * 16, 16)])


@functools.partial(
    pl.kernel,
    out_type=jax.ShapeDtypeStruct((NOUT * C,), jnp.float32),
    mesh=_mesh,
    scratch_types=[
        pltpu.VMEM((ROWS * C,), jnp.float32),
        pltpu.VMEM((KPAD,), jnp.int32),
        pltpu.VMEM((CAP,), jnp.int32),
        pltpu.VMEM((CAP,), jnp.int32),
        pltpu.VMEM((G,), jnp.int32),
        pltpu.VMEM((G,), jnp.int32),
        pltpu.VMEM((G, C), jnp.float32),
        pltpu.VMEM((G, C), jnp.float32),
        pltpu.VMEM((NW * 16 + 16,), jnp.int32),
        pltpu.VMEM_SHARED((NKEYS,), jnp.int32),
        pltpu.SemaphoreType.DMA,
        pltpu.SemaphoreType.DMA,
    ],
    compiler_params=_sc_params,
)
def _scatter_kernel(feat_hbm, pk_hbm, bounds_hbm, out_hbm, acc, vbuf, mid,
                    mkey, idxg0, idxg1, rows0, rows1, bvmem, kshared, gsem0,
                    gsem1):
  w = _wid()
  lanes = jnp.arange(16, dtype=jnp.int32)
  neg = jnp.full((16,), -jnp.inf, dtype=jnp.float32)
  idxgs = (idxg0, idxg1)
  rowss = (rows0, rows1)
  gsems = (gsem0, gsem1)

  @pl.when(lax.axis_index("s") == 0)
  def _():
    pltpu.sync_copy(pk_hbm, kshared)

  pltpu.sync_copy(bounds_hbm, bvmem.at[pl.ds(0, NW * 16)])
  plsc.subcore_barrier()

  def gissue(start, buf):
    for u in range(G // 16):
      idxgs[buf][pl.ds(u * 16, 16)] = mid[pl.ds(start + u * 16, 16)]
    pltpu.async_copy(feat_hbm.at[idxgs[buf]], rowss[buf], gsems[buf])

  def gwait(buf):
    pltpu.make_async_copy(feat_hbm.at[idxgs[buf]], rowss[buf],
                          gsems[buf]).wait()

  def rmw_rows(buf, start, nj, lo):
    rows = rowss[buf]

    def rmw(j, _):
      kj = mkey[pl.ds(start + j, 16)][0]
      off = (kj - lo) * C
      for u in range(C // 16):
        a = acc[pl.ds(off + u * 16, 16)]
        f = rows[j, pl.ds(u * 16, 16)]
        acc[pl.ds(off + u * 16, 16)] = jnp.maximum(a, f)
      return 0

    lax.fori_loop(0, nj, rmw, 0)

  def drain_full(cnt, done, lo):
    nfull = (cnt - done) // G

    def one(b, d):
      gissue(d, 0)
      gwait(0)
      rmw_rows(0, d, jnp.int32(G), lo)
      return d + G

    return lax.fori_loop(0, nfull, one, done)

  def do_pass(p, _):
    rid = p * NW + w
    lo = rid * ROWS
    hi = lo + ROWS
    plo = lo << 12
    phi = hi << 12

    def initb(i, _):
      for u in range(8):
        acc[pl.ds(i * 128 + u * 16, 16)] = neg
      return 0

    lax.fori_loop(0, ROWS * C // 128, initb, 0)

    def per_v(v, carry):
      cnt, done = carry
      done = drain_full(cnt, done, lo)
      for u in range(9):
        mid[pl.ds(u * 16, 16)] = mid[pl.ds(done + u * 16, 16)]
        mkey[pl.ds(u * 16, 16)] = mkey[pl.ds(done + u * 16, 16)]
      cnt = cnt - done
      s = bvmem[pl.ds(v * 16 + p, 16)][0]
      e = bvmem[pl.ds(v * 16 + p + 1, 16)][0]
      blk0 = s // BLK
      nblk = (e + BLK - 1) // BLK - blk0

      def cpb(bb, _):
        pltpu.sync_copy(
            kshared.at[pl.ds(v * KPAD + (blk0 + bb) * BLK, BLK)],
            vbuf.at[pl.ds(bb * BLK, BLK)])
        return 0

      lax.fori_loop(0, nblk, cpb, 0)
      base = blk0 * BLK
      i0 = (s - base) // 16
      i1 = (e - base + 15) // 16
      vb = v * PTS_W

      def scan(i, c):
        pk = vbuf[pl.ds(i * 16, 16)]
        gpos = base + i * 16 + lanes
        m = (pk >= plo) & (pk < phi) & (gpos >= s) & (gpos < e)
        mi = m.astype(jnp.int32)
        pos = plsc.cumsum(mi) - 1 + c
        plsc.store_scatter(mkey, [pos], pk >> 12, mask=m)
        plsc.store_scatter(mid, [pos], vb + (pk & 0xFFF), mask=m)
        return c + jnp.sum(mi, axis=0)

      cnt = lax.fori_loop(i0, i1, scan, cnt)
      return (cnt, jnp.int32(0))

    cnt, done = lax.fori_loop(0, NW, per_v, (jnp.int32(0), jnp.int32(0)))
    done = drain_full(cnt, done, lo)
    nrem = cnt - done

    @pl.when(nrem > 0)
    def _():
      iv = jnp.full((16,), mid[pl.ds(done, 16)][0], jnp.int32)
      kv = jnp.full((16,), mkey[pl.ds(done, 16)][0], jnp.int32)
      for u in range(G // 16):
        mid[pl.ds(cnt + u * 16, 16)] = iv
        mkey[pl.ds(cnt + u * 16, 16)] = kv
      gissue(done, 0)
      gwait(0)
      rmw_rows(0, done, nrem, lo)

    pltpu.sync_copy(acc, out_hbm.at[pl.ds(rid * (ROWS * C), ROWS * C)])
    return 0

  lax.fori_loop(0, NPASS, do_pass, 0)


def _fixup_body(x_ref, o_ref):
  x = x_ref[0]
  xt = x.T
  o_ref[0] = jnp.where(xt == -jnp.inf, jnp.float32(0.0), xt)


_fixup = pl.pallas_call(
    _fixup_body,
    grid=(BATCH,),
    in_specs=[pl.BlockSpec((1, OY * OX, C), lambda b: (b, 0, 0))],
    out_specs=pl.BlockSpec((1, C, OY * OX), lambda b: (b, 0, 0)),
    out_shape=jax.ShapeDtypeStruct((BATCH, C, OY * OX), jnp.float32),
)


def kernel(features, coors, batch_size):
  del batch_size
  coflat = jnp.reshape(coors.astype(jnp.int32), (-1,))
  coflat = jnp.concatenate([coflat, jnp.zeros((32,), jnp.int32)])
  pk, bounds = _keys_kernel(coflat)
  accflat = _scatter_kernel(features, pk, bounds)
  dense = _fixup(accflat.reshape(BATCH, OY * OX, C))
  return dense.reshape(BATCH, C, OY, OX)
